# baseline (speedup 1.0000x reference)
import jax
import jax.numpy as jnp
from jax.experimental import pallas as pl

_B, _T = 8, 12000
_D, _K = 64, 8192
_L = 1000
_N = _B * _L
_BM = 400
_GRID = _N // _BM
_U = 256

_HI = jax.lax.Precision.HIGHEST


def _vq_stats_block(z_ref, cb_ref, idx_ref, avg_ref, cnt_ref, com_ref):
    i = pl.program_id(0)
    z = z_ref[...]
    cb = cb_ref[...]
    idx = idx_ref[0, 0, :]
    zz = jnp.sum(z * z, axis=1, keepdims=True)
    cc = jnp.sum(cb * cb, axis=1)[None, :]
    zc = jax.lax.dot_general(z, cb, (((1,), (1,)), ((), ())),
                             preferred_element_type=jnp.float32,
                             precision=_HI)
    d2 = zz - 2.0 * zc + cc
    m = jnp.min(d2, axis=1, keepdims=True)
    e = jnp.exp((m - d2) * 0.01)
    s = jnp.sum(e, axis=1, keepdims=True)
    probs = e / s
    onehot = (jax.lax.broadcasted_iota(jnp.int32, (_BM, _K), 1)
              == idx[:, None]).astype(jnp.float32)
    q = jax.lax.dot_general(onehot, cb, (((1,), (0,)), ((), ())),
                            preferred_element_type=jnp.float32,
                            precision=_HI)

    @pl.when(i == 0)
    def _init():
        avg_ref[...] = jnp.zeros_like(avg_ref)
        cnt_ref[...] = jnp.zeros_like(cnt_ref)
        com_ref[...] = jnp.zeros_like(com_ref)

    avg_ref[...] += jnp.sum(probs, axis=0)[None, :]
    cnt_ref[...] += jnp.sum(onehot, axis=0)[None, :]
    com_ref[...] += jnp.sum((z - q) ** 2)[None, None]


def _loss_epilogue(cb_ref, uids_ref, avg_ref, com_ref,
                   loss_ref, commit_ref, div_ref, ortho_ref):
    cb = cb_ref[...]
    uids = uids_ref[0, :].astype(jnp.int32)
    onehot = (jax.lax.broadcasted_iota(jnp.int32, (_U, _K), 1)
              == uids[:, None]).astype(jnp.float32)
    codes = jax.lax.dot_general(onehot, cb, (((1,), (0,)), ((), ())),
                                preferred_element_type=jnp.float32,
                                precision=_HI)
    nrm = jnp.sqrt(jnp.sum(codes * codes, axis=1, keepdims=True)) + 1e-8
    normed = codes / nrm
    cs = jax.lax.dot_general(normed, normed, (((1,), (1,)), ((), ())),
                             preferred_element_type=jnp.float32,
                             precision=_HI)
    ortho = jnp.sum(cs * cs) / float(_U * _U) - 1.0 / _U
    avg = avg_ref[...] * (1.0 / _N)
    div = jnp.sum(avg * jnp.log(avg + 1e-10))
    commit = jnp.sum(com_ref[...]) * (1.0 / (_N * _D))
    loss_ref[...] = (commit + div + ortho)[None, None]
    commit_ref[...] = commit[None, None]
    div_ref[...] = div[None, None]
    ortho_ref[...] = ortho[None, None]


def _vq_losses(zf, codebook, idx):
    f32 = jnp.float32
    idx3 = idx.reshape(_GRID, 1, _BM)
    avg, cnt, com = pl.pallas_call(
        _vq_stats_block,
        grid=(_GRID,),
        in_specs=[
            pl.BlockSpec((_BM, _D), lambda i: (i, 0)),
            pl.BlockSpec((_K, _D), lambda i: (0, 0)),
            pl.BlockSpec((1, 1, _BM), lambda i: (i, 0, 0)),
        ],
        out_specs=[
            pl.BlockSpec((1, _K), lambda i: (0, 0)),
            pl.BlockSpec((1, _K), lambda i: (0, 0)),
            pl.BlockSpec((1, 1), lambda i: (0, 0)),
        ],
        out_shape=[
            jax.ShapeDtypeStruct((1, _K), f32),
            jax.ShapeDtypeStruct((1, _K), f32),
            jax.ShapeDtypeStruct((1, 1), f32),
        ],
    )(zf, codebook, idx3)

    uids = jnp.nonzero(cnt[0] > 0.0, size=_U, fill_value=0)[0]
    uids_f = uids.astype(f32)[None, :]

    loss, commit, div, ortho = pl.pallas_call(
        _loss_epilogue,
        in_specs=[
            pl.BlockSpec((_K, _D), lambda: (0, 0)),
            pl.BlockSpec((1, _U), lambda: (0, 0)),
            pl.BlockSpec((1, _K), lambda: (0, 0)),
            pl.BlockSpec((1, 1), lambda: (0, 0)),
        ],
        out_specs=[pl.BlockSpec((1, 1), lambda: (0, 0))] * 4,
        out_shape=[jax.ShapeDtypeStruct((1, 1), f32)] * 4,
    )(codebook, uids_f, avg, com)

    return (loss.reshape(()), commit.reshape(()),
            div.reshape(()), ortho.reshape(()))


def _conv1d(x, w, b, stride, pad):
    y = jax.lax.conv_general_dilated(
        x, w, (stride,), [(pad, pad)],
        dimension_numbers=('NCH', 'OIH', 'NCH'))
    if b is not None:
        y = y + b[None, :, None]
    return y


def _conv_transpose1d(x, w, stride, pad):
    k = w.shape[2]
    wk = jnp.flip(w, axis=2).transpose(1, 0, 2)
    return jax.lax.conv_general_dilated(
        x, wk, (1,), [(k - 1 - pad, k - 1 - pad)], lhs_dilation=(stride,),
        dimension_numbers=('NCH', 'OIH', 'NCH'))


def _bnorm(x, g, b, eps=1e-5):
    m = jnp.mean(x, axis=(0, 2), keepdims=True)
    v = jnp.var(x, axis=(0, 2), keepdims=True)
    return g[None, :, None] * (x - m) / jnp.sqrt(v + eps) + b[None, :, None]


def _silu(x):
    return x * jax.nn.sigmoid(x)


def kernel(x, enc_w1, enc_b1, bn1_g, bn1_b, enc_w2, enc_b2, bn2_g, bn2_b,
           enc_w3, enc_b3, bn3_g, bn3_b, codebook, dec_wt, bn4_g, bn4_b,
           dec_w2, dec_b2, bn5_g, bn5_b, dec_w3, dec_b3):
    h = _conv1d(x, enc_w1, enc_b1, 1, 2)
    h = _silu(h)
    h = _bnorm(h, bn1_g, bn1_b)
    h = _conv1d(h, enc_w2, enc_b2, 1, 2)
    h = _silu(h)
    h = _bnorm(h, bn2_g, bn2_b)
    h = _conv1d(h, enc_w3, enc_b3, 12, 12)
    h = jnp.tanh(h)
    h = _bnorm(h, bn3_g, bn3_b)
    z = h.transpose(0, 2, 1)

    zf = z.reshape(-1, _D)
    d2 = (jnp.sum(zf ** 2, axis=1, keepdims=True)
          - 2.0 * zf @ codebook.T
          + jnp.sum(codebook ** 2, axis=1)[None, :])
    idx = jnp.argmin(d2, axis=1)
    q = codebook[idx]
    q_st = zf + jax.lax.stop_gradient(q - zf)

    loss, commit, div, ortho = _vq_losses(zf, codebook, idx)

    zq = q_st.reshape(_B, _L, _D).transpose(0, 2, 1)
    r = _conv_transpose1d(zq, dec_wt, 12, 12)
    r = _silu(r)
    r = _bnorm(r, bn4_g, bn4_b)
    r = _conv1d(r, dec_w2, dec_b2, 1, 2)
    r = _silu(r)
    r = _bnorm(r, bn5_g, bn5_b)
    r = _conv1d(r, dec_w3, dec_b3, 1, 0)
    tl = x.shape[2]
    cl = r.shape[2]
    if cl > tl:
        r = r[:, :, :tl]
    elif cl < tl:
        r = jnp.pad(r, ((0, 0), (0, 0), (0, tl - cl)))
    return (r, idx.reshape(_B, _L), loss, commit, div, ortho)


# speedup vs baseline: 1.4867x; 1.4867x over previous
import jax
import jax.numpy as jnp
from jax.experimental import pallas as pl

_B, _T = 8, 12000
_D, _K = 64, 8192
_L = 1000
_N = _B * _L
_BM = 400
_GRID = _N // _BM
_U = 256

_HI = jax.lax.Precision.HIGHEST


def _vq_stats_block(z_ref, cb_ref, idx_ref, q_ref, avg_ref, cnt_ref, com_ref):
    i = pl.program_id(0)
    z = z_ref[...]
    cb = cb_ref[...]
    idx = idx_ref[0, 0, :]
    q = q_ref[...]
    zz = jnp.sum(z * z, axis=1, keepdims=True)
    cc = jnp.sum(cb * cb, axis=1)[None, :]
    zc = jax.lax.dot_general(z, cb, (((1,), (1,)), ((), ())),
                             preferred_element_type=jnp.float32)
    d2 = zz - 2.0 * zc + cc
    m = jnp.min(d2, axis=1, keepdims=True)
    e = jnp.exp((m - d2) * 0.01)
    s = jnp.sum(e, axis=1, keepdims=True)
    probs = e / s
    onehot = (jax.lax.broadcasted_iota(jnp.int32, (_BM, _K), 1)
              == idx[:, None]).astype(jnp.float32)

    @pl.when(i == 0)
    def _init():
        avg_ref[...] = jnp.zeros_like(avg_ref)
        cnt_ref[...] = jnp.zeros_like(cnt_ref)
        com_ref[...] = jnp.zeros_like(com_ref)

    avg_ref[...] += jnp.sum(probs, axis=0)[None, :]
    cnt_ref[...] += jnp.sum(onehot, axis=0)[None, :]
    com_ref[...] += jnp.sum((z - q) ** 2)[None, None]


def _loss_epilogue(cb_ref, uids_ref, avg_ref, com_ref,
                   loss_ref, commit_ref, div_ref, ortho_ref):
    cb = cb_ref[...]
    uids = uids_ref[0, :].astype(jnp.int32)
    onehot = (jax.lax.broadcasted_iota(jnp.int32, (_U, _K), 1)
              == uids[:, None]).astype(jnp.float32)
    codes = jax.lax.dot_general(onehot, cb, (((1,), (0,)), ((), ())),
                                preferred_element_type=jnp.float32,
                                precision=_HI)
    nrm = jnp.sqrt(jnp.sum(codes * codes, axis=1, keepdims=True)) + 1e-8
    normed = codes / nrm
    cs = jax.lax.dot_general(normed, normed, (((1,), (1,)), ((), ())),
                             preferred_element_type=jnp.float32,
                             precision=_HI)
    ortho = jnp.sum(cs * cs) / float(_U * _U) - 1.0 / _U
    avg = avg_ref[...] * (1.0 / _N)
    div = jnp.sum(avg * jnp.log(avg + 1e-10))
    commit = jnp.sum(com_ref[...]) * (1.0 / (_N * _D))
    loss_ref[...] = (commit + div + ortho)[None, None]
    commit_ref[...] = commit[None, None]
    div_ref[...] = div[None, None]
    ortho_ref[...] = ortho[None, None]


def _vq_losses(zf, codebook, idx, q):
    f32 = jnp.float32
    idx3 = idx.reshape(_GRID, 1, _BM)
    avg, cnt, com = pl.pallas_call(
        _vq_stats_block,
        grid=(_GRID,),
        in_specs=[
            pl.BlockSpec((_BM, _D), lambda i: (i, 0)),
            pl.BlockSpec((_K, _D), lambda i: (0, 0)),
            pl.BlockSpec((1, 1, _BM), lambda i: (i, 0, 0)),
            pl.BlockSpec((_BM, _D), lambda i: (i, 0)),
        ],
        out_specs=[
            pl.BlockSpec((1, _K), lambda i: (0, 0)),
            pl.BlockSpec((1, _K), lambda i: (0, 0)),
            pl.BlockSpec((1, 1), lambda i: (0, 0)),
        ],
        out_shape=[
            jax.ShapeDtypeStruct((1, _K), f32),
            jax.ShapeDtypeStruct((1, _K), f32),
            jax.ShapeDtypeStruct((1, 1), f32),
        ],
    )(zf, codebook, idx3, q)

    uids = jnp.nonzero(cnt[0] > 0.0, size=_U, fill_value=0)[0]
    uids_f = uids.astype(f32)[None, :]

    loss, commit, div, ortho = pl.pallas_call(
        _loss_epilogue,
        in_specs=[
            pl.BlockSpec((_K, _D), lambda: (0, 0)),
            pl.BlockSpec((1, _U), lambda: (0, 0)),
            pl.BlockSpec((1, _K), lambda: (0, 0)),
            pl.BlockSpec((1, 1), lambda: (0, 0)),
        ],
        out_specs=[pl.BlockSpec((1, 1), lambda: (0, 0))] * 4,
        out_shape=[jax.ShapeDtypeStruct((1, 1), f32)] * 4,
    )(codebook, uids_f, avg, com)

    return (loss.reshape(()), commit.reshape(()),
            div.reshape(()), ortho.reshape(()))


def _conv1d(x, w, b, stride, pad):
    y = jax.lax.conv_general_dilated(
        x, w, (stride,), [(pad, pad)],
        dimension_numbers=('NCH', 'OIH', 'NCH'))
    if b is not None:
        y = y + b[None, :, None]
    return y


def _conv_transpose1d(x, w, stride, pad):
    k = w.shape[2]
    wk = jnp.flip(w, axis=2).transpose(1, 0, 2)
    return jax.lax.conv_general_dilated(
        x, wk, (1,), [(k - 1 - pad, k - 1 - pad)], lhs_dilation=(stride,),
        dimension_numbers=('NCH', 'OIH', 'NCH'))


def _bnorm(x, g, b, eps=1e-5):
    m = jnp.mean(x, axis=(0, 2), keepdims=True)
    v = jnp.var(x, axis=(0, 2), keepdims=True)
    return g[None, :, None] * (x - m) / jnp.sqrt(v + eps) + b[None, :, None]


def _silu(x):
    return x * jax.nn.sigmoid(x)


def kernel(x, enc_w1, enc_b1, bn1_g, bn1_b, enc_w2, enc_b2, bn2_g, bn2_b,
           enc_w3, enc_b3, bn3_g, bn3_b, codebook, dec_wt, bn4_g, bn4_b,
           dec_w2, dec_b2, bn5_g, bn5_b, dec_w3, dec_b3):
    h = _conv1d(x, enc_w1, enc_b1, 1, 2)
    h = _silu(h)
    h = _bnorm(h, bn1_g, bn1_b)
    h = _conv1d(h, enc_w2, enc_b2, 1, 2)
    h = _silu(h)
    h = _bnorm(h, bn2_g, bn2_b)
    h = _conv1d(h, enc_w3, enc_b3, 12, 12)
    h = jnp.tanh(h)
    h = _bnorm(h, bn3_g, bn3_b)
    z = h.transpose(0, 2, 1)

    zf = z.reshape(-1, _D)
    d2 = (jnp.sum(zf ** 2, axis=1, keepdims=True)
          - 2.0 * zf @ codebook.T
          + jnp.sum(codebook ** 2, axis=1)[None, :])
    idx = jnp.argmin(d2, axis=1)
    q = codebook[idx]
    q_st = zf + jax.lax.stop_gradient(q - zf)

    loss, commit, div, ortho = _vq_losses(zf, codebook, idx, q)

    zq = q_st.reshape(_B, _L, _D).transpose(0, 2, 1)
    r = _conv_transpose1d(zq, dec_wt, 12, 12)
    r = _silu(r)
    r = _bnorm(r, bn4_g, bn4_b)
    r = _conv1d(r, dec_w2, dec_b2, 1, 2)
    r = _silu(r)
    r = _bnorm(r, bn5_g, bn5_b)
    r = _conv1d(r, dec_w3, dec_b3, 1, 0)
    tl = x.shape[2]
    cl = r.shape[2]
    if cl > tl:
        r = r[:, :, :tl]
    elif cl < tl:
        r = jnp.pad(r, ((0, 0), (0, 0), (0, tl - cl)))
    return (r, idx.reshape(_B, _L), loss, commit, div, ortho)
